# baseline (speedup 1.0000x reference)
import functools

import jax
import jax.numpy as jnp
from jax import lax
from jax.experimental import pallas as pl
from jax.experimental.pallas import tpu as pltpu
from jax.experimental.pallas import tpu_sc as plsc

_VOCAB1 = 1000001

_N = 16384 * 200
_NW = 32
_PER_W = _N // _NW
_CHUNK = 12800
_NCH = _PER_W // _CHUNK
_VPC = _CHUNK // 16
_UNROLL = 4

_MESH = plsc.VectorSubcoreMesh(core_axis_name="c", subcore_axis_name="s")


def _log_vreg(x):
    bits = lax.bitcast_convert_type(x, jnp.int32)
    e = (bits >> 23) - 126
    mbits = (bits & 0x007FFFFF) | 0x3F000000
    m = lax.bitcast_convert_type(mbits, jnp.float32)
    below = m < 0.70710678
    m = jnp.where(below, m + m, m)
    e = jnp.where(below, e - 1, e)
    xf = m - 1.0
    z = xf * xf
    y = jnp.float32(7.0376836292e-2)
    for c in (-1.1514610310e-1, 1.1676998740e-1, -1.2420140846e-1,
              1.4249322787e-1, -1.6668057665e-1, 2.0000714765e-1,
              -2.4999993993e-1, 3.3333331174e-1):
        y = y * xf + jnp.float32(c)
    y = xf * z * y
    ef = e.astype(jnp.float32)
    y = y + ef * jnp.float32(-2.12194440e-4)
    y = y - jnp.float32(0.5) * z
    r = xf + y + ef * jnp.float32(0.693359375)
    return jnp.where(x == 0.0, -jnp.inf, r)


def _log_chunk(val_ref):
    def body(k, _):
        for u in range(_UNROLL):
            sl = pl.ds(k * (16 * _UNROLL) + u * 16, 16)
            val_ref[sl] = _log_vreg(val_ref[sl])
        return 0

    lax.fori_loop(0, _VPC // _UNROLL, body, 0)


@functools.partial(
    pl.kernel,
    out_type=jax.ShapeDtypeStruct((_N,), jnp.float32),
    mesh=_MESH,
    scratch_types=[
        pltpu.VMEM((_CHUNK,), jnp.int32),
        pltpu.VMEM((_CHUNK,), jnp.int32),
        pltpu.VMEM((_CHUNK,), jnp.float32),
        pltpu.VMEM((_CHUNK,), jnp.float32),
        pltpu.VMEM_SHARED((_VOCAB1,), jnp.float32),
        pltpu.SemaphoreType.DMA,
        pltpu.SemaphoreType.DMA,
    ],
)
def _sc_kernel(tab_hbm, idx_hbm, out_hbm, idx0, idx1, val0, val1,
               tab_sp, sem0, sem1):
    sid = lax.axis_index("s")
    wid = sid * 2 + lax.axis_index("c")
    base = wid * _PER_W

    @pl.when(sid == 0)
    def _():
        pltpu.sync_copy(tab_hbm, tab_sp)

    plsc.subcore_barrier()

    bufs = ((idx0, val0, sem0), (idx1, val1, sem1))

    def load_and_fire(i):
        ib, vb, sm = bufs[i % 2]
        pltpu.sync_copy(idx_hbm.at[pl.ds(base + i * _CHUNK, _CHUNK)], ib)
        return pltpu.async_copy(tab_sp.at[ib], vb, sm)

    descs = [None, None]
    descs[0] = load_and_fire(0)
    for i in range(_NCH):
        if i + 1 < _NCH:
            descs[(i + 1) % 2] = load_and_fire(i + 1)
        _, vb, _ = bufs[i % 2]
        descs[i % 2].wait()
        _log_chunk(vb)
        pltpu.sync_copy(vb, out_hbm.at[pl.ds(base + i * _CHUNK, _CHUNK)])


def kernel(pop_prob, items):
    idx = items.reshape(-1).astype(jnp.int32)
    out = _sc_kernel(pop_prob, idx)
    return out.reshape(items.shape)


# speedup vs baseline: 1.2846x; 1.2846x over previous
import functools

import jax
import jax.numpy as jnp
from jax import lax
from jax.experimental import pallas as pl
from jax.experimental.pallas import tpu as pltpu
from jax.experimental.pallas import tpu_sc as plsc

_VOCAB1 = 1000001
_VOCAB_PAD = 1000448
_ROWS = _VOCAB_PAD // 128

_N = 16384 * 200
_NW = 32
_PER_W = _N // _NW
_CHUNK = 12800
_NCH = _PER_W // _CHUNK


def _log_body(p_ref, o_ref):
    o_ref[...] = jnp.log(p_ref[...])


def _log_table(table_2d):
    return pl.pallas_call(
        _log_body,
        out_shape=jax.ShapeDtypeStruct((_ROWS, 128), jnp.float32),
    )(table_2d)


_MESH = plsc.VectorSubcoreMesh(core_axis_name="c", subcore_axis_name="s")


@functools.partial(
    pl.kernel,
    out_type=jax.ShapeDtypeStruct((_N,), jnp.float32),
    mesh=_MESH,
    scratch_types=[
        pltpu.VMEM((_CHUNK,), jnp.int32),
        pltpu.VMEM((_CHUNK,), jnp.int32),
        pltpu.VMEM((_CHUNK,), jnp.float32),
        pltpu.VMEM((_CHUNK,), jnp.float32),
        pltpu.VMEM_SHARED((_VOCAB_PAD,), jnp.float32),
        pltpu.SemaphoreType.DMA,
        pltpu.SemaphoreType.DMA,
    ],
)
def _sc_gather(table_hbm, idx_hbm, out_hbm, idx0, idx1, val0, val1,
               tab_sp, sem0, sem1):
    sid = lax.axis_index("s")
    wid = sid * 2 + lax.axis_index("c")
    base = wid * _PER_W

    @pl.when(sid == 0)
    def _():
        pltpu.sync_copy(table_hbm, tab_sp)

    plsc.subcore_barrier()

    bufs = ((idx0, val0, sem0), (idx1, val1, sem1))

    def load_and_fire(i):
        ib, vb, sm = bufs[i % 2]
        pltpu.sync_copy(idx_hbm.at[pl.ds(base + i * _CHUNK, _CHUNK)], ib)
        return pltpu.async_copy(tab_sp.at[ib], vb, sm)

    descs = [None, None]
    descs[0] = load_and_fire(0)
    for i in range(_NCH):
        if i + 1 < _NCH:
            descs[(i + 1) % 2] = load_and_fire(i + 1)
        _, vb, _ = bufs[i % 2]
        descs[i % 2].wait()
        pltpu.sync_copy(vb, out_hbm.at[pl.ds(base + i * _CHUNK, _CHUNK)])


def kernel(pop_prob, items):
    table = jnp.pad(pop_prob, (0, _VOCAB_PAD - _VOCAB1), constant_values=1.0)
    logt = _log_table(table.reshape(_ROWS, 128)).reshape(-1)
    idx = items.reshape(-1).astype(jnp.int32)
    out = _sc_gather(logt, idx)
    return out.reshape(items.shape)
